# baseline (speedup 1.0000x reference)
import jax
import jax.numpy as jnp
from jax import lax
from jax.experimental import pallas as pl
from jax.experimental.pallas import tpu as pltpu
from jax.experimental.pallas import tpu_sc as plsc

B = 16384
D = 64
L = 16
NC, NS = 2, 16
NW = NC * NS
BPW = B // NW
CH = 16
NCHUNK = BPW // CH


def _user_body(uid_hbm, ut_hbm, uemb_hbm,
               idx_u, sub_u, rows_u, ebuf, sems):
    wid = lax.axis_index("s") * NC + lax.axis_index("c")
    base = wid * BPW

    pltpu.sync_copy(uid_hbm.at[pl.ds(base, BPW)], idx_u)

    def prep(j, carry):
        s = pl.ds(j * L, L)
        sub_u[s] = jnp.bitwise_and(idx_u[s], 7)
        return carry

    lax.fori_loop(0, BPW // L, prep, 0)

    lane = lax.iota(jnp.int32, L)
    ut3 = ut_hbm.reshape(ut_hbm.shape[0] // 8, 8, D)

    def issue(c, buf):
        tu_vec = lax.shift_right_logical(idx_u[pl.ds(c * CH, CH)], 3)
        for j in range(CH):
            pltpu.async_copy(ut3.at[pl.ds(tu_vec[j], 1)],
                             rows_u.at[buf, pl.ds(j, 1)], sems.at[buf])

    def drain(buf):
        def one(j, carry):
            pltpu.make_async_copy(ut3.at[pl.ds(0, 1)],
                                  rows_u.at[0, pl.ds(0, 1)],
                                  sems.at[buf]).wait()
            return carry
        lax.fori_loop(0, CH, one, 0)

    def chunk_body(c, carry):
        buf = jnp.bitwise_and(c, 1)
        nbuf = jnp.bitwise_and(c + 1, 1)

        @pl.when(c + 1 < NCHUNK)
        def _():
            issue(c + 1, nbuf)

        drain(buf)

        bufv = jnp.broadcast_to(buf, (L,))
        row = lane
        s = pl.ds(c * CH, L)
        su = sub_u[s]
        erow = c * CH + lane
        for d in range(D):
            dcol = jnp.bitwise_and(lane + d, D - 1)
            u = plsc.load_gather(rows_u, [bufv, row, su, dcol])
            plsc.store_scatter(ebuf, [erow, dcol], u)
        return carry

    issue(0, 0)
    lax.fori_loop(0, NCHUNK, chunk_body, 0)
    pltpu.sync_copy(ebuf, uemb_hbm.at[pl.ds(base, BPW)])


def _item_body(iid_hbm, it_hbm, uemb_hbm, out_hbm,
               idx_i, rows_u, rows_i, out_v, sem_u, sem_i):
    wid = lax.axis_index("s") * NC + lax.axis_index("c")
    base = wid * BPW

    pltpu.sync_copy(iid_hbm.at[pl.ds(base, BPW)], idx_i)
    cu = pltpu.async_copy(uemb_hbm.at[pl.ds(base, BPW)], rows_u, sem_u)
    ci = pltpu.async_copy(it_hbm.at[idx_i], rows_i, sem_i)
    cu.wait()
    ci.wait()

    lane = lax.iota(jnp.int32, L)

    def blk_body(blk, carry):
        row = blk * L + lane
        s = pl.ds(blk * L, L)
        acc = jnp.zeros((L,), jnp.float32)
        for d in range(D):
            dcol = jnp.bitwise_and(lane + d, D - 1)
            u = plsc.load_gather(rows_u, [row, dcol])
            v = plsc.load_gather(rows_i, [row, dcol])
            acc = acc + u * v
        out_v[s] = acc
        return carry

    lax.fori_loop(0, BPW // L, blk_body, 0)
    pltpu.sync_copy(out_v, out_hbm.at[pl.ds(base, BPW)])


@jax.jit
def kernel(user_ids, item_ids, user_table, item_table):
    mesh = plsc.VectorSubcoreMesh(core_axis_name="c", subcore_axis_name="s",
                                  num_cores=NC, num_subcores=NS)
    user_stage = pl.kernel(
        _user_body,
        out_type=jax.ShapeDtypeStruct((B, D), jnp.float32),
        mesh=mesh,
        compiler_params=pltpu.CompilerParams(needs_layout_passes=False,
                                             use_tc_tiling_on_sc=True),
        scratch_types=[
            pltpu.VMEM((BPW,), jnp.int32),
            pltpu.VMEM((BPW,), jnp.int32),
            pltpu.VMEM((2, CH, 8, D), jnp.float32),
            pltpu.VMEM((BPW, D), jnp.float32),
            pltpu.SemaphoreType.DMA((2,)),
        ],
    )
    dot_stage = pl.kernel(
        _item_body,
        out_type=jax.ShapeDtypeStruct((B,), jnp.float32),
        mesh=mesh,
        compiler_params=pltpu.CompilerParams(needs_layout_passes=False,
                                             use_tc_tiling_on_sc=False),
        scratch_types=[
            pltpu.VMEM((BPW,), jnp.int32),
            pltpu.VMEM((BPW, D), jnp.float32),
            pltpu.VMEM((BPW, D), jnp.float32),
            pltpu.VMEM((BPW,), jnp.float32),
            pltpu.SemaphoreType.DMA,
            pltpu.SemaphoreType.DMA,
        ],
    )
    u_emb = user_stage(user_ids, user_table)
    return dot_stage(item_ids, item_table, u_emb)


# speedup vs baseline: 1.0088x; 1.0088x over previous
import jax
import jax.numpy as jnp
from jax import lax
from jax.experimental import pallas as pl
from jax.experimental.pallas import tpu as pltpu
from jax.experimental.pallas import tpu_sc as plsc

B = 16384
D = 64
L = 16
NC, NS = 2, 16
NW = NC * NS
BPW = B // NW
CH = 16
NCHUNK = BPW // CH


def _user_body(uid_hbm, ut_hbm, uemb_hbm,
               idx_u, sub_u, rows_u, ebuf, sems):
    wid = lax.axis_index("s") * NC + lax.axis_index("c")
    base = wid * BPW

    pltpu.sync_copy(uid_hbm.at[pl.ds(base, BPW)], idx_u)

    def prep(j, carry):
        s = pl.ds(j * L, L)
        sub_u[s] = jnp.bitwise_and(idx_u[s], 7)
        return carry

    lax.fori_loop(0, BPW // L, prep, 0)

    lane = lax.iota(jnp.int32, L)
    ut3 = ut_hbm.reshape(ut_hbm.shape[0] // 8, 8, D)

    def issue(c, buf):
        tu_vec = lax.shift_right_logical(idx_u[pl.ds(c * CH, CH)], 3)
        for j in range(CH):
            pltpu.async_copy(ut3.at[pl.ds(tu_vec[j], 1)],
                             rows_u.at[buf, pl.ds(j, 1)], sems.at[buf])

    def drain(buf):
        def one(j, carry):
            pltpu.make_async_copy(ut3.at[pl.ds(0, 1)],
                                  rows_u.at[0, pl.ds(0, 1)],
                                  sems.at[buf]).wait()
            return carry
        lax.fori_loop(0, CH, one, 0)

    def chunk_body(c, carry):
        buf = jnp.bitwise_and(c, 1)
        nbuf = jnp.bitwise_and(c + 1, 1)

        @pl.when(c + 1 < NCHUNK)
        def _():
            issue(c + 1, nbuf)

        drain(buf)

        bufv = jnp.broadcast_to(buf, (L,))
        row = lane
        s = pl.ds(c * CH, L)
        su = sub_u[s]
        erow = c * CH + lane
        for d in range(D):
            dcol = jnp.bitwise_and(lane + d, D - 1)
            u = plsc.load_gather(rows_u, [bufv, row, su, dcol])
            plsc.store_scatter(ebuf, [erow * D + dcol], u)
        return carry

    issue(0, 0)
    lax.fori_loop(0, NCHUNK, chunk_body, 0)
    pltpu.sync_copy(ebuf, uemb_hbm.at[pl.ds(base * D, BPW * D)])


def _item_body(iid_hbm, it_hbm, uemb_hbm, out_hbm,
               idx_i, rows_u, rows_i, out_v, sem_u, sem_i):
    wid = lax.axis_index("s") * NC + lax.axis_index("c")
    base = wid * BPW

    pltpu.sync_copy(iid_hbm.at[pl.ds(base, BPW)], idx_i)
    cu = pltpu.async_copy(uemb_hbm.at[pl.ds(base * D, BPW * D)], rows_u, sem_u)
    ci = pltpu.async_copy(it_hbm.at[idx_i], rows_i, sem_i)
    cu.wait()
    ci.wait()

    lane = lax.iota(jnp.int32, L)

    def blk_body(blk, carry):
        row = blk * L + lane
        s = pl.ds(blk * L, L)
        acc = jnp.zeros((L,), jnp.float32)
        for d in range(D):
            dcol = jnp.bitwise_and(lane + d, D - 1)
            u = plsc.load_gather(rows_u, [row * D + dcol])
            v = plsc.load_gather(rows_i, [row, dcol])
            acc = acc + u * v
        out_v[s] = acc
        return carry

    lax.fori_loop(0, BPW // L, blk_body, 0)
    pltpu.sync_copy(out_v, out_hbm.at[pl.ds(base, BPW)])


@jax.jit
def kernel(user_ids, item_ids, user_table, item_table):
    mesh = plsc.VectorSubcoreMesh(core_axis_name="c", subcore_axis_name="s",
                                  num_cores=NC, num_subcores=NS)
    user_stage = pl.kernel(
        _user_body,
        out_type=jax.ShapeDtypeStruct((B * D,), jnp.float32),
        mesh=mesh,
        compiler_params=pltpu.CompilerParams(needs_layout_passes=False,
                                             use_tc_tiling_on_sc=True),
        scratch_types=[
            pltpu.VMEM((BPW,), jnp.int32),
            pltpu.VMEM((BPW,), jnp.int32),
            pltpu.VMEM((2, CH, 8, D), jnp.float32),
            pltpu.VMEM((BPW * D,), jnp.float32),
            pltpu.SemaphoreType.DMA((2,)),
        ],
    )
    dot_stage = pl.kernel(
        _item_body,
        out_type=jax.ShapeDtypeStruct((B,), jnp.float32),
        mesh=mesh,
        compiler_params=pltpu.CompilerParams(needs_layout_passes=False,
                                             use_tc_tiling_on_sc=False),
        scratch_types=[
            pltpu.VMEM((BPW,), jnp.int32),
            pltpu.VMEM((BPW * D,), jnp.float32),
            pltpu.VMEM((BPW, D), jnp.float32),
            pltpu.VMEM((BPW,), jnp.float32),
            pltpu.SemaphoreType.DMA,
            pltpu.SemaphoreType.DMA,
        ],
    )
    u_emb = user_stage(user_ids, user_table)
    return dot_stage(item_ids, item_table, u_emb)


# speedup vs baseline: 1.2142x; 1.2036x over previous
import jax
import jax.numpy as jnp
from jax import lax
from jax.experimental import pallas as pl
from jax.experimental.pallas import tpu as pltpu
from jax.experimental.pallas import tpu_sc as plsc

B = 16384
D = 64
L = 16
NC, NS = 2, 16
NW = NC * NS
BPW = B // NW
CH = 16
NCHUNK = BPW // CH


def _user_body(uid_hbm, ut_hbm, uemb_hbm,
               idx_u, sub_u, rows_u, ebuf, sems):
    wid = lax.axis_index("s") * NC + lax.axis_index("c")
    base = wid * BPW

    pltpu.sync_copy(uid_hbm.at[pl.ds(base, BPW)], idx_u)

    def prep(j, carry):
        s = pl.ds(j * L, L)
        sub_u[s] = jnp.bitwise_and(idx_u[s], 7)
        return carry

    lax.fori_loop(0, BPW // L, prep, 0)

    lane = lax.iota(jnp.int32, L)
    ut3 = ut_hbm.reshape(ut_hbm.shape[0] // 8, 8, D)

    def issue(c, buf):
        tu_vec = lax.shift_right_logical(idx_u[pl.ds(c * CH, CH)], 3)
        for j in range(CH):
            pltpu.async_copy(ut3.at[pl.ds(tu_vec[j], 1)],
                             rows_u.at[buf, pl.ds(j, 1)], sems.at[buf])

    def drain(buf):
        def one(j, carry):
            pltpu.make_async_copy(ut3.at[pl.ds(0, 1)],
                                  rows_u.at[0, pl.ds(0, 1)],
                                  sems.at[buf]).wait()
            return carry
        lax.fori_loop(0, CH, one, 0)

    def chunk_body(c, carry):
        buf = jnp.bitwise_and(c, 1)
        nbuf = jnp.bitwise_and(c + 1, 1)

        @pl.when(c + 1 < NCHUNK)
        def _():
            issue(c + 1, nbuf)

        drain(buf)

        bufv = jnp.broadcast_to(buf, (L,))
        row = lane
        s = pl.ds(c * CH, L)
        su = sub_u[s]
        erow = c * CH + lane
        for d in range(D):
            dcol = jnp.bitwise_and(lane + d, D - 1)
            u = plsc.load_gather(rows_u, [bufv, row, su, dcol])
            plsc.store_scatter(ebuf, [erow * D + dcol], u)
        return carry

    issue(0, 0)
    lax.fori_loop(0, NCHUNK, chunk_body, 0)
    pltpu.sync_copy(ebuf, uemb_hbm.at[pl.ds(base * D, BPW * D)])


def _item_body(iid_hbm, it_hbm, uemb_hbm, out_hbm,
               idx_i, sub_i, rows_i, ubuf, out_v, sems, sem_u):
    wid = lax.axis_index("s") * NC + lax.axis_index("c")
    base = wid * BPW

    pltpu.sync_copy(iid_hbm.at[pl.ds(base, BPW)], idx_i)
    cu = pltpu.async_copy(uemb_hbm.at[pl.ds(base * D, BPW * D)], ubuf, sem_u)

    def prep(j, carry):
        s = pl.ds(j * L, L)
        sub_i[s] = jnp.bitwise_and(idx_i[s], 7)
        return carry

    lax.fori_loop(0, BPW // L, prep, 0)
    cu.wait()

    lane = lax.iota(jnp.int32, L)
    it3 = it_hbm.reshape(it_hbm.shape[0] // 8, 8, D)

    def issue(c, buf):
        ti_vec = lax.shift_right_logical(idx_i[pl.ds(c * CH, CH)], 3)
        for j in range(CH):
            pltpu.async_copy(it3.at[pl.ds(ti_vec[j], 1)],
                             rows_i.at[buf, pl.ds(j, 1)], sems.at[buf])

    def drain(buf):
        def one(j, carry):
            pltpu.make_async_copy(it3.at[pl.ds(0, 1)],
                                  rows_i.at[0, pl.ds(0, 1)],
                                  sems.at[buf]).wait()
            return carry
        lax.fori_loop(0, CH, one, 0)

    def chunk_body(c, carry):
        buf = jnp.bitwise_and(c, 1)
        nbuf = jnp.bitwise_and(c + 1, 1)

        @pl.when(c + 1 < NCHUNK)
        def _():
            issue(c + 1, nbuf)

        drain(buf)

        bufv = jnp.broadcast_to(buf, (L,))
        row = lane
        s = pl.ds(c * CH, L)
        si = sub_i[s]
        erow = c * CH + lane
        acc = jnp.zeros((L,), jnp.float32)
        for d in range(D):
            dcol = jnp.bitwise_and(lane + d, D - 1)
            v = plsc.load_gather(rows_i, [bufv, row, si, dcol])
            u = plsc.load_gather(ubuf, [erow * D + dcol])
            acc = acc + u * v
        out_v[s] = acc
        return carry

    issue(0, 0)
    lax.fori_loop(0, NCHUNK, chunk_body, 0)
    pltpu.sync_copy(out_v, out_hbm.at[pl.ds(base, BPW)])


@jax.jit
def kernel(user_ids, item_ids, user_table, item_table):
    mesh = plsc.VectorSubcoreMesh(core_axis_name="c", subcore_axis_name="s",
                                  num_cores=NC, num_subcores=NS)
    user_stage = pl.kernel(
        _user_body,
        out_type=jax.ShapeDtypeStruct((B * D,), jnp.float32),
        mesh=mesh,
        compiler_params=pltpu.CompilerParams(needs_layout_passes=False,
                                             use_tc_tiling_on_sc=True),
        scratch_types=[
            pltpu.VMEM((BPW,), jnp.int32),
            pltpu.VMEM((BPW,), jnp.int32),
            pltpu.VMEM((2, CH, 8, D), jnp.float32),
            pltpu.VMEM((BPW * D,), jnp.float32),
            pltpu.SemaphoreType.DMA((2,)),
        ],
    )
    dot_stage = pl.kernel(
        _item_body,
        out_type=jax.ShapeDtypeStruct((B,), jnp.float32),
        mesh=mesh,
        compiler_params=pltpu.CompilerParams(needs_layout_passes=False,
                                             use_tc_tiling_on_sc=True),
        scratch_types=[
            pltpu.VMEM((BPW,), jnp.int32),
            pltpu.VMEM((BPW,), jnp.int32),
            pltpu.VMEM((2, CH, 8, D), jnp.float32),
            pltpu.VMEM((BPW * D,), jnp.float32),
            pltpu.VMEM((BPW,), jnp.float32),
            pltpu.SemaphoreType.DMA((2,)),
            pltpu.SemaphoreType.DMA,
        ],
    )
    u_emb = user_stage(user_ids, user_table)
    return dot_stage(item_ids, item_table, u_emb)
